# Initial kernel scaffold; baseline (speedup 1.0000x reference)
import jax
import jax.numpy as jnp
from jax.experimental import pallas as pl


def kernel(x_car, x_pedestrian, x_traffic, edge_index_car, edge_index_pedestrian, edge_index_traffic, W1_car, b1_car, W2_car, b2_car, W1_pedestrian, b1_pedestrian, W2_pedestrian, b2_pedestrian, W1_traffic, b1_traffic, W2_traffic, b2_traffic):
    raise NotImplementedError("write your pallas kernel here")


# speedup vs baseline: 3.4644x; 3.4644x over previous
import functools

import jax
import jax.numpy as jnp
from jax import lax
from jax.experimental import pallas as pl
from jax.experimental.pallas import tpu as pltpu
from jax.experimental.pallas import tpu_sc as plsc

N = 10000
E = 640000
D = 128
NC = 2
NS = 16
CH = 128
NROWS = 5120
EPAD = NROWS * CH
TPT = NROWS // (NC * NS)
ACC_ROWS = 10240
NB = 2
RB = 1024
NBLK = ACC_ROWS // RB

_mesh = plsc.VectorSubcoreMesh(
    core_axis_name="c", subcore_axis_name="s", num_cores=NC, num_subcores=NS)

_f32 = jnp.float32


def _deg_body(d0, d1, d2, o0, o1, o2, dbuf, hist):
    core = lax.axis_index("c")
    sid = lax.axis_index("s")
    wid = core * NS + sid
    ones = jnp.full((16,), 1.0, _f32)

    for d_hbm, o_hbm in ((d0, o0), (d1, o1), (d2, o2)):
        def _z(i, carry):
            hist[pl.ds(i * 16, 16)] = jnp.zeros((16,), _f32)
            return carry
        lax.fori_loop(0, ACC_ROWS // 16, _z, 0)

        row0 = core * (NS * TPT) + sid * TPT
        pltpu.sync_copy(d_hbm.at[pl.ds(row0, TPT)], dbuf)

        def _r(r, carry):
            for k in range(CH // 16):
                iv = dbuf[r, pl.ds(k * 16, 16)]
                plsc.addupdate_scatter(hist, [iv], ones)
            return carry
        lax.fori_loop(0, TPT, _r, 0)

        pltpu.sync_copy(hist, o_hbm.at[pl.ds(wid * ACC_ROWS, ACC_ROWS)])


_deg_call = pl.kernel(
    _deg_body,
    out_type=(jax.ShapeDtypeStruct((NC * NS * ACC_ROWS,), _f32),) * 3,
    mesh=_mesh,
    compiler_params=pltpu.CompilerParams(needs_layout_passes=False),
    scratch_types=[
        pltpu.VMEM((TPT, CH), jnp.int32),
        pltpu.VMEM((ACC_ROWS,), _f32),
    ],
)


SG = 16
NSG = TPT // SG


def _seg_body(h, s2, d2, z, out, sbuf, dbuf, rbuf, acc,
              g0, g1, t0, t1, i0, i1):
    gs = (g0, g1)
    ts = (t0, t1)
    core = lax.axis_index("c")
    sid = lax.axis_index("s")

    pltpu.sync_copy(z, rbuf.at[0])
    for k in range(5):
        pltpu.sync_copy(rbuf.at[0], acc.at[pl.ds(sid * 640 + k * CH, CH)])
    plsc.subcore_barrier()

    row0 = core * (NS * TPT) + sid * TPT
    pltpu.sync_copy(s2.at[pl.ds(row0, SG)], sbuf.at[0])
    pltpu.sync_copy(d2.at[pl.ds(row0, SG)], dbuf.at[0])

    for sg in range(NSG):
        pb = sg % 2
        sb = sbuf.at[pb]
        db = dbuf.at[pb]
        nxt = row0 + (sg + 1) * SG
        for j in range(2):
            pltpu.async_copy(h.at[sb.at[j]], rbuf.at[j], gs[j])
        if sg < NSG - 1:
            pltpu.async_copy(s2.at[pl.ds(nxt, SG)], sbuf.at[1 - pb], i0)
            pltpu.async_copy(d2.at[pl.ds(nxt, SG)], dbuf.at[1 - pb], i1)

        def _grp(g, carry):
            for j in range(2):
                c = g * 2 + j
                pltpu.make_async_copy(h.at[sb.at[c]], rbuf.at[j], gs[j]).wait()
                pltpu.async_copy(rbuf.at[j], acc.at[db.at[c]], ts[j], add=True)

            @pl.when(g < SG // 2 - 1)
            def _next():
                for j in range(2):
                    c = g * 2 + j
                    pltpu.make_async_copy(
                        rbuf.at[j], acc.at[db.at[c]], ts[j]).wait()
                    pltpu.async_copy(h.at[sb.at[c + 2]], rbuf.at[j], gs[j])
            return carry
        lax.fori_loop(0, SG // 2, _grp, 0)

        for j in range(2):
            pltpu.make_async_copy(rbuf.at[j], acc.at[db.at[0]], ts[j]).wait()
        if sg < NSG - 1:
            pltpu.make_async_copy(s2.at[pl.ds(nxt, SG)], sbuf.at[1 - pb], i0).wait()
            pltpu.make_async_copy(d2.at[pl.ds(nxt, SG)], dbuf.at[1 - pb], i1).wait()

    plsc.subcore_barrier()

    for k in range(5):
        r = sid * 640 + k * CH
        pltpu.sync_copy(acc.at[pl.ds(r, CH)], rbuf.at[0])
        pltpu.sync_copy(rbuf.at[0], out.at[core, pl.ds(r, CH)])


_seg_call = pl.kernel(
    _seg_body,
    out_type=jax.ShapeDtypeStruct((NC, ACC_ROWS, D), _f32),
    mesh=_mesh,
    scratch_types=[
        pltpu.VMEM((2, SG, CH), jnp.int32),
        pltpu.VMEM((2, SG, CH), jnp.int32),
        pltpu.VMEM((2, CH, D), _f32),
        pltpu.VMEM_SHARED((ACC_ROWS, D), _f32),
    ] + [pltpu.SemaphoreType.DMA] * 6,
)


def _norm_from(dr):
    deg = jnp.sum(dr[...], axis=0, keepdims=True)
    degc = jnp.transpose(deg, (1, 0))
    return jnp.where(degc > 0.0, lax.rsqrt(jnp.maximum(degc, 1.0)), 0.0)


def _prep_body(x0, x1, x2, dp0, dp1, dp2, h0, h1, h2):
    for xr, dr, hr in ((x0, dp0, h0), (x1, dp1, h1), (x2, dp2, h2)):
        hr[...] = xr[...] * _norm_from(dr)


_x_spec = pl.BlockSpec((RB, D), lambda i: (i, 0))
_dp_spec = pl.BlockSpec((NC * NS, RB), lambda i: (0, i))
_agg_spec = pl.BlockSpec((NC, RB, D), lambda i: (0, i, 0))
_w_spec = pl.BlockSpec((D, D), lambda i: (0, 0))
_b_spec = pl.BlockSpec((1, D), lambda i: (0, 0))

_prep_call = pl.pallas_call(
    _prep_body,
    grid=(NBLK,),
    in_specs=[_x_spec] * 3 + [_dp_spec] * 3,
    out_specs=[_x_spec] * 3,
    out_shape=[jax.ShapeDtypeStruct((N, D), _f32)] * 3,
)


def _lin_body(scale_out, a0, a1, a2, dp0, dp1, dp2, w0, w1, w2,
              b0, b1, b2, o0, o1, o2):
    for ar, dr, wr, br, orf in ((a0, dp0, w0, b0, o0),
                                (a1, dp1, w1, b1, o1),
                                (a2, dp2, w2, b2, o2)):
        norm = _norm_from(dr)
        agg = (ar[0] + ar[1]) * norm
        y = lax.dot_general(agg, wr[...], (((1,), (1,)), ((), ())),
                            preferred_element_type=_f32) + br[...]
        y = jnp.maximum(y, 0.0)
        orf[...] = y * norm if scale_out else y


def _make_lin(scale_out):
    return pl.pallas_call(
        functools.partial(_lin_body, scale_out),
        grid=(NBLK,),
        in_specs=[_agg_spec] * 3 + [_dp_spec] * 3 + [_w_spec] * 3 + [_b_spec] * 3,
        out_specs=[_x_spec] * 3,
        out_shape=[jax.ShapeDtypeStruct((N, D), _f32)] * 3,
    )


_lin_scaled = _make_lin(True)
_lin_plain = _make_lin(False)


def _prep_edges(e):
    pad = EPAD - E
    src = jnp.concatenate([e[0], jnp.zeros((pad,), jnp.int32)]).reshape(NROWS, CH)
    dst = jnp.concatenate([e[1], jnp.full((pad,), N, jnp.int32)]).reshape(NROWS, CH)
    return src, dst


def kernel(x_car, x_pedestrian, x_traffic,
           edge_index_car, edge_index_pedestrian, edge_index_traffic,
           W1_car, b1_car, W2_car, b2_car,
           W1_pedestrian, b1_pedestrian, W2_pedestrian, b2_pedestrian,
           W1_traffic, b1_traffic, W2_traffic, b2_traffic):
    s_c, d_c = _prep_edges(edge_index_car)
    s_p, d_p = _prep_edges(edge_index_pedestrian)
    s_t, d_t = _prep_edges(edge_index_traffic)
    z = jnp.zeros((CH, D), _f32)

    dg_c, dg_p, dg_t = _deg_call(d_c, d_p, d_t)
    dg_c = dg_c.reshape(NC * NS, ACC_ROWS)
    dg_p = dg_p.reshape(NC * NS, ACC_ROWS)
    dg_t = dg_t.reshape(NC * NS, ACC_ROWS)

    h0_c, h0_p, h0_t = _prep_call(x_car, x_pedestrian, x_traffic,
                                  dg_c, dg_p, dg_t)

    a1_c = _seg_call(h0_c, s_c, d_c, z)
    a1_p = _seg_call(h0_p, s_p, d_p, z)
    a1_t = _seg_call(h0_t, s_t, d_t, z)

    b1 = (b1_car.reshape(1, D), b1_pedestrian.reshape(1, D),
          b1_traffic.reshape(1, D))
    h1_c, h1_p, h1_t = _lin_scaled(a1_c, a1_p, a1_t, dg_c, dg_p, dg_t,
                                   W1_car, W1_pedestrian, W1_traffic, *b1)

    a2_c = _seg_call(h1_c, s_c, d_c, z)
    a2_p = _seg_call(h1_p, s_p, d_p, z)
    a2_t = _seg_call(h1_t, s_t, d_t, z)

    b2 = (b2_car.reshape(1, D), b2_pedestrian.reshape(1, D),
          b2_traffic.reshape(1, D))
    out_c, out_p, out_t = _lin_plain(a2_c, a2_p, a2_t, dg_c, dg_p, dg_t,
                                     W2_car, W2_pedestrian, W2_traffic, *b2)
    return (out_c, out_p, out_t)
